# baseline (speedup 1.0000x reference)
import functools

import jax
import jax.numpy as jnp
from jax import lax
from jax.experimental import pallas as pl
from jax.experimental.pallas import tpu as pltpu
from jax.experimental.pallas import tpu_sc as plsc

B = 16384
D = 64
NC = 2
NS = 16
NW = NC * NS
CHUNK = 128
CHUNKS_PER_W = B // (NW * CHUNK)
ROWS_PER_W = B // NW
LANES = 16


def _tanh16(x):
    return 1.0 - 2.0 / (1.0 + jnp.exp(x * 2.0))


def _body(idx_hbm, table_hbm, out_hbm, idx_v, rows_v, sems):
    wid = lax.axis_index("s") * NC + lax.axis_index("c")
    chunk0 = wid * CHUNKS_PER_W

    pltpu.sync_copy(idx_hbm.at[pl.ds(chunk0, CHUNKS_PER_W)], idx_v)

    copies = []
    for j in range(CHUNKS_PER_W):
        copies.append(
            pltpu.async_copy(table_hbm.at[idx_v.at[j]], rows_v.at[j], sems.at[j])
        )

    for j in range(CHUNKS_PER_W):
        copies[j].wait()

        def compute_row(i, _, j=j):
            for c in range(D // LANES):
                x = rows_v[j, i, pl.ds(c * LANES, LANES)]
                rows_v[j, i, pl.ds(c * LANES, LANES)] = _tanh16(x)
            return 0

        lax.fori_loop(0, CHUNK, compute_row, 0)

        pltpu.sync_copy(
            rows_v.at[j], out_hbm.at[pl.ds((chunk0 + j) * CHUNK, CHUNK)]
        )


@functools.partial(jax.jit, static_argnames=())
def _run(topics2d, table):
    mesh = plsc.VectorSubcoreMesh(core_axis_name="c", subcore_axis_name="s")
    f = functools.partial(
        pl.kernel,
        out_type=jax.ShapeDtypeStruct((B, D), jnp.float32),
        mesh=mesh,
        scratch_types=[
            pltpu.VMEM((CHUNKS_PER_W, CHUNK), jnp.int32),
            pltpu.VMEM((CHUNKS_PER_W, CHUNK, D), jnp.float32),
            pltpu.SemaphoreType.DMA((CHUNKS_PER_W,)),
        ],
        compiler_params=pltpu.CompilerParams(use_tc_tiling_on_sc=False),
    )(_body)
    return f(topics2d, table)


def kernel(topics, table):
    topics2d = topics.astype(jnp.int32).reshape(B // CHUNK, CHUNK)
    out = _run(topics2d, table)
    return out.reshape(B, 1, D)


# speedup vs baseline: 1.7187x; 1.7187x over previous
import functools

import jax
import jax.numpy as jnp
from jax import lax
from jax.experimental import pallas as pl
from jax.experimental.pallas import tpu as pltpu
from jax.experimental.pallas import tpu_sc as plsc

B = 16384
D = 64
NC = 2
NS = 16
NW = NC * NS
CHUNK = 128
CHUNKS_PER_W = B // (NW * CHUNK)
LANES = 16


def _tanh16(x):
    return 1.0 - 2.0 / (1.0 + jnp.exp(x * 2.0))


def _body(idx_hbm, table_hbm, out_hbm, idx_v, rows_v, sems):
    wid = lax.axis_index("s") * NC + lax.axis_index("c")
    chunk0 = wid * CHUNKS_PER_W

    pltpu.sync_copy(idx_hbm.at[pl.ds(chunk0, CHUNKS_PER_W)], idx_v)

    for j in range(CHUNKS_PER_W):
        def issue_rows(g, _, j=j):
            idx16 = idx_v[j, pl.ds(g * LANES, LANES)]
            for lane in range(LANES):
                r = idx16[lane]
                pltpu.make_async_copy(
                    table_hbm.at[pl.ds(r, 1)],
                    rows_v.at[j, pl.ds(g * LANES + lane, 1)],
                    sems.at[j],
                ).start()
            return 0

        lax.fori_loop(0, CHUNK // LANES, issue_rows, 0)

    for j in range(CHUNKS_PER_W):
        pltpu.make_async_copy(
            table_hbm.at[pl.ds(0, CHUNK)], rows_v.at[j], sems.at[j]
        ).wait()

        def compute_row(i, _, j=j):
            for c in range(D // LANES):
                x = rows_v[j, i, pl.ds(c * LANES, LANES)]
                rows_v[j, i, pl.ds(c * LANES, LANES)] = _tanh16(x)
            return 0

        lax.fori_loop(0, CHUNK, compute_row, 0)

        pltpu.sync_copy(
            rows_v.at[j], out_hbm.at[pl.ds((chunk0 + j) * CHUNK, CHUNK)]
        )


@jax.jit
def _run(topics2d, table):
    mesh = plsc.VectorSubcoreMesh(core_axis_name="c", subcore_axis_name="s")
    f = functools.partial(
        pl.kernel,
        out_type=jax.ShapeDtypeStruct((B, D), jnp.float32),
        mesh=mesh,
        scratch_types=[
            pltpu.VMEM((CHUNKS_PER_W, CHUNK), jnp.int32),
            pltpu.VMEM((CHUNKS_PER_W, CHUNK, D), jnp.float32),
            pltpu.SemaphoreType.DMA((CHUNKS_PER_W,)),
        ],
    )(_body)
    return f(topics2d, table)


def kernel(topics, table):
    topics2d = topics.astype(jnp.int32).reshape(B // CHUNK, CHUNK)
    out = _run(topics2d, table)
    return out.reshape(B, 1, D)


# speedup vs baseline: 2.2393x; 1.3029x over previous
import functools

import jax
import jax.numpy as jnp
from jax import lax
from jax.experimental import pallas as pl
from jax.experimental.pallas import tpu as pltpu
from jax.experimental.pallas import tpu_sc as plsc

B = 16384
D = 64
NC = 2
NS = 16
NW = NC * NS
CHUNK = 128
CHUNKS_PER_W = B // (NW * CHUNK)
LANES = 16
NBUF = 8


def _tanh16(x):
    return 1.0 - 2.0 / (1.0 + jnp.exp(x * 2.0))


def _body(idx_hbm, table_t_hbm, out_t_hbm, idx_v, bufs_v, out_v, sems):
    wid = lax.axis_index("s") * NC + lax.axis_index("c")
    chunk0 = wid * CHUNKS_PER_W
    iota = lax.iota(jnp.int32, LANES)

    pltpu.sync_copy(idx_hbm.at[pl.ds(chunk0, CHUNKS_PER_W)], idx_v)

    def issue(s, r):
        off = pl.multiple_of((r >> 7) * CHUNK, CHUNK)
        pltpu.make_async_copy(
            table_t_hbm.at[:, pl.ds(off, CHUNK)], bufs_v.at[s], sems.at[s]
        ).start()

    def extract(s, r, col):
        m = jnp.full((LANES,), r & 127, jnp.int32)
        colv = jnp.full((LANES,), col, jnp.int32)
        for c0 in range(0, D, LANES):
            x = plsc.load_gather(bufs_v.at[s], [iota + c0, m])
            plsc.store_scatter(out_v, [iota + c0, colv], _tanh16(x))

    def drain(s):
        pltpu.make_async_copy(
            table_t_hbm.at[:, pl.ds(0, CHUNK)], bufs_v.at[s], sems.at[s]
        ).wait()

    for j in range(CHUNKS_PER_W):
        def group_body(h, idx_prev, j=j):
            idx16 = idx_v[j, pl.ds(h * LANES, LANES)]

            @pl.when(h > 0)
            def _():
                for s in range(NBUF):
                    drain(s)
                    extract(s, idx_prev[NBUF + s], (h - 1) * LANES + NBUF + s)

            for s in range(NBUF):
                issue(s, idx16[s])
            for s in range(NBUF):
                drain(s)
                extract(s, idx16[s], h * LANES + s)
            for s in range(NBUF):
                issue(s, idx16[NBUF + s])
            return idx16

        idx_last = lax.fori_loop(
            0, CHUNK // LANES, group_body, jnp.zeros((LANES,), jnp.int32)
        )
        for s in range(NBUF):
            drain(s)
            extract(s, idx_last[NBUF + s], CHUNK - NBUF + s)

        colbase = pl.multiple_of((chunk0 + j) * CHUNK, CHUNK)
        pltpu.sync_copy(out_v, out_t_hbm.at[:, pl.ds(colbase, CHUNK)])


@jax.jit
def _run(topics2d, table_t):
    mesh = plsc.VectorSubcoreMesh(core_axis_name="c", subcore_axis_name="s")
    f = functools.partial(
        pl.kernel,
        out_type=jax.ShapeDtypeStruct((D, B), jnp.float32),
        mesh=mesh,
        scratch_types=[
            pltpu.VMEM((CHUNKS_PER_W, CHUNK), jnp.int32),
            pltpu.VMEM((NBUF, D, CHUNK), jnp.float32),
            pltpu.VMEM((D, CHUNK), jnp.float32),
            pltpu.SemaphoreType.DMA((NBUF,)),
        ],
        compiler_params=pltpu.CompilerParams(needs_layout_passes=False),
    )(_body)
    return f(topics2d, table_t)


def kernel(topics, table):
    topics2d = topics.astype(jnp.int32).reshape(B // CHUNK, CHUNK)
    out_t = _run(topics2d, table.T)
    return out_t.T.reshape(B, 1, D)


# speedup vs baseline: 2.2512x; 1.0053x over previous
import functools

import jax
import jax.numpy as jnp
from jax import lax
from jax.experimental import pallas as pl
from jax.experimental.pallas import tpu as pltpu
from jax.experimental.pallas import tpu_sc as plsc

B = 16384
D = 64
NC = 2
NS = 16
NW = NC * NS
CHUNK = 128
CHUNKS_PER_W = B // (NW * CHUNK)
LANES = 16
NBUF = 8


def _tanh16(x):
    return 1.0 - 2.0 / (1.0 + jnp.exp(x * 2.0))


def _body(idx_hbm, table_t_hbm, out_t_hbm, idx_v, bufs_v, out_v, sems):
    wid = lax.axis_index("s") * NC + lax.axis_index("c")
    chunk0 = wid * CHUNKS_PER_W
    iota = lax.iota(jnp.int32, LANES)

    pltpu.sync_copy(idx_hbm.at[pl.ds(chunk0, CHUNKS_PER_W)], idx_v)

    def issue(s, r):
        off = pl.multiple_of((r >> 7) * CHUNK, CHUNK)
        for a in range(0, D, 8):
            pltpu.make_async_copy(
                table_t_hbm.at[pl.ds(a, 8), pl.ds(off, CHUNK)],
                bufs_v.at[s, pl.ds(a, 8)],
                sems.at[s],
            ).start()

    def extract(s, r, col):
        m = jnp.full((LANES,), r & 127, jnp.int32)
        colv = jnp.full((LANES,), col, jnp.int32)
        for c0 in range(0, D, LANES):
            x = plsc.load_gather(bufs_v.at[s], [iota + c0, m])
            plsc.store_scatter(out_v, [iota + c0, colv], _tanh16(x))

    def drain(s):
        pltpu.make_async_copy(
            table_t_hbm.at[:, pl.ds(0, CHUNK)], bufs_v.at[s], sems.at[s]
        ).wait()

    for j in range(CHUNKS_PER_W):
        def group_body(h, idx_prev, j=j):
            idx16 = idx_v[j, pl.ds(h * LANES, LANES)]

            @pl.when(h > 0)
            def _():
                for s in range(NBUF):
                    drain(s)
                    extract(s, idx_prev[NBUF + s], (h - 1) * LANES + NBUF + s)

            for s in range(NBUF):
                issue(s, idx16[s])
            for s in range(NBUF):
                drain(s)
                extract(s, idx16[s], h * LANES + s)
            for s in range(NBUF):
                issue(s, idx16[NBUF + s])
            return idx16

        idx_last = lax.fori_loop(
            0, CHUNK // LANES, group_body, jnp.zeros((LANES,), jnp.int32)
        )
        for s in range(NBUF):
            drain(s)
            extract(s, idx_last[NBUF + s], CHUNK - NBUF + s)

        colbase = pl.multiple_of((chunk0 + j) * CHUNK, CHUNK)
        pltpu.sync_copy(out_v, out_t_hbm.at[:, pl.ds(colbase, CHUNK)])


@jax.jit
def _run(topics2d, table_t):
    mesh = plsc.VectorSubcoreMesh(core_axis_name="c", subcore_axis_name="s")
    f = functools.partial(
        pl.kernel,
        out_type=jax.ShapeDtypeStruct((D, B), jnp.float32),
        mesh=mesh,
        scratch_types=[
            pltpu.VMEM((CHUNKS_PER_W, CHUNK), jnp.int32),
            pltpu.VMEM((NBUF, D, CHUNK), jnp.float32),
            pltpu.VMEM((D, CHUNK), jnp.float32),
            pltpu.SemaphoreType.DMA((NBUF,)),
        ],
        compiler_params=pltpu.CompilerParams(needs_layout_passes=False),
    )(_body)
    return f(topics2d, table_t)


def kernel(topics, table):
    topics2d = topics.astype(jnp.int32).reshape(B // CHUNK, CHUNK)
    out_t = _run(topics2d, table.T)
    return out_t.T.reshape(B, 1, D)


# speedup vs baseline: 2.8117x; 1.2490x over previous
import functools

import jax
import jax.numpy as jnp
from jax import lax
from jax.experimental import pallas as pl
from jax.experimental.pallas import tpu as pltpu
from jax.experimental.pallas import tpu_sc as plsc

B = 16384
D = 64
V = 1000000
NC = 2
NS = 16
NW = NC * NS
CHUNK = 128
LANES = 16
NBUF = 8

B_TC = 8192
B_SC = B - B_TC
CHUNKS_PER_W = B_SC // (NW * CHUNK)
TC_BATCH = 128
TC_NBATCH = B_TC // TC_BATCH


def _tanh16(x):
    return 1.0 - 2.0 / (1.0 + jnp.exp(x * 2.0))


def _sc_body(idx_hbm, table_t_hbm, out_t_hbm, idx_v, bufs_v, out_v, sems):
    wid = lax.axis_index("s") * NC + lax.axis_index("c")
    chunk0 = wid * CHUNKS_PER_W
    iota = lax.iota(jnp.int32, LANES)

    pltpu.sync_copy(idx_hbm.at[pl.ds(chunk0, CHUNKS_PER_W)], idx_v)

    def issue(s, r):
        off = pl.multiple_of((r >> 7) * CHUNK, CHUNK)
        for a in range(0, D, 8):
            pltpu.make_async_copy(
                table_t_hbm.at[pl.ds(a, 8), pl.ds(off, CHUNK)],
                bufs_v.at[s, pl.ds(a, 8)],
                sems.at[s],
            ).start()

    def extract(s, r, col):
        m = jnp.full((LANES,), r & 127, jnp.int32)
        colv = jnp.full((LANES,), col, jnp.int32)
        for c0 in range(0, D, LANES):
            x = plsc.load_gather(bufs_v.at[s], [iota + c0, m])
            plsc.store_scatter(out_v, [iota + c0, colv], _tanh16(x))

    def drain(s):
        pltpu.make_async_copy(
            table_t_hbm.at[:, pl.ds(0, CHUNK)], bufs_v.at[s], sems.at[s]
        ).wait()

    for j in range(CHUNKS_PER_W):
        def group_body(h, idx_prev, j=j):
            idx16 = idx_v[j, pl.ds(h * LANES, LANES)]

            @pl.when(h > 0)
            def _():
                for s in range(NBUF):
                    drain(s)
                    extract(s, idx_prev[NBUF + s], (h - 1) * LANES + NBUF + s)

            for s in range(NBUF):
                issue(s, idx16[s])
            for s in range(NBUF):
                drain(s)
                extract(s, idx16[s], h * LANES + s)
            for s in range(NBUF):
                issue(s, idx16[NBUF + s])
            return idx16

        idx_last = lax.fori_loop(
            0, CHUNK // LANES, group_body, jnp.zeros((LANES,), jnp.int32)
        )
        for s in range(NBUF):
            drain(s)
            extract(s, idx_last[NBUF + s], CHUNK - NBUF + s)

        colbase = pl.multiple_of((chunk0 + j) * CHUNK, CHUNK)
        pltpu.sync_copy(out_v, out_t_hbm.at[:, pl.ds(colbase, CHUNK)])


def _run_sc(topics2d, table_t):
    mesh = plsc.VectorSubcoreMesh(core_axis_name="c", subcore_axis_name="s")
    f = functools.partial(
        pl.kernel,
        out_type=jax.ShapeDtypeStruct((D, B_SC), jnp.float32),
        mesh=mesh,
        scratch_types=[
            pltpu.VMEM((CHUNKS_PER_W, CHUNK), jnp.int32),
            pltpu.VMEM((NBUF, D, CHUNK), jnp.float32),
            pltpu.VMEM((D, CHUNK), jnp.float32),
            pltpu.SemaphoreType.DMA((NBUF,)),
        ],
        compiler_params=pltpu.CompilerParams(needs_layout_passes=False),
    )(_sc_body)
    return f(topics2d, table_t)


def _tc_body(idx_s, idx_v_ref, table_t_hbm, tail_hbm, out_ref, bufs_v, sems):
    lane_iota = lax.broadcasted_iota(jnp.int32, (TC_BATCH, CHUNK), 1)
    last_q = (V - 1) >> 7

    def issue_batch(k, kmod):
        def issue_one(i, _):
            r = idx_s[k * TC_BATCH + i]
            q = r >> 7
            off = pl.multiple_of(q * CHUNK, CHUNK)

            @pl.when(q < last_q)
            def _():
                pltpu.make_async_copy(
                    table_t_hbm.at[:, pl.ds(off, CHUNK)],
                    bufs_v.at[kmod, i],
                    sems.at[kmod],
                ).start()

            @pl.when(q >= last_q)
            def _():
                pltpu.make_async_copy(
                    tail_hbm, bufs_v.at[kmod, i], sems.at[kmod]
                ).start()

            return 0

        lax.fori_loop(0, TC_BATCH, issue_one, 0)

    def drain_batch(kmod):
        def drain_one(i, _):
            pltpu.make_async_copy(
                table_t_hbm.at[:, pl.ds(0, CHUNK)],
                bufs_v.at[kmod, i],
                sems.at[kmod],
            ).wait()
            return 0

        lax.fori_loop(0, TC_BATCH, drain_one, 0)

    issue_batch(0, 0)

    def batch_body(k, _):
        kmod = lax.rem(k, 2)

        @pl.when(k < TC_NBATCH - 1)
        def _():
            issue_batch(k + 1, lax.rem(k + 1, 2))

        drain_batch(kmod)
        g = bufs_v[kmod]
        m = idx_v_ref[pl.ds(k * TC_BATCH, TC_BATCH)] & 127
        onehot = (lane_iota == m[:, None]).astype(jnp.float32)
        e = jnp.sum(g * onehot[:, None, :], axis=2)
        out_ref[:, pl.ds(k * TC_BATCH, TC_BATCH)] = jnp.tanh(e).T
        return 0

    lax.fori_loop(0, TC_NBATCH, batch_body, 0)


def _run_tc(topics_tc, table_t, tail):
    return pl.pallas_call(
        _tc_body,
        out_shape=jax.ShapeDtypeStruct((D, B_TC), jnp.float32),
        in_specs=[
            pl.BlockSpec(memory_space=pltpu.SMEM),
            pl.BlockSpec(memory_space=pltpu.VMEM),
            pl.BlockSpec(memory_space=pl.ANY),
            pl.BlockSpec(memory_space=pl.ANY),
        ],
        out_specs=pl.BlockSpec(memory_space=pltpu.VMEM),
        scratch_shapes=[
            pltpu.VMEM((2, TC_BATCH, D, CHUNK), jnp.float32),
            pltpu.SemaphoreType.DMA((2,)),
        ],
    )(topics_tc, topics_tc, table_t, tail)


@jax.jit
def _run(topics, table_t):
    topics_sc = topics[B_TC:].reshape(B_SC // CHUNK, CHUNK)
    last = (V >> 7) * CHUNK
    tail = jnp.pad(table_t[:, last:], ((0, 0), (0, CHUNK - (V - last))))
    out_sc = _run_sc(topics_sc, table_t)
    out_tc = _run_tc(topics[:B_TC], table_t, tail)
    return jnp.concatenate([out_tc, out_sc], axis=1)


def kernel(topics, table):
    out_t = _run(topics.astype(jnp.int32), table.T)
    return out_t.T.reshape(B, 1, D)


# speedup vs baseline: 3.2393x; 1.1521x over previous
import functools

import jax
import jax.numpy as jnp
from jax import lax
from jax.experimental import pallas as pl
from jax.experimental.pallas import tpu as pltpu
from jax.experimental.pallas import tpu_sc as plsc

B = 16384
D = 64
V = 1000000
NC = 2
NS = 16
NW = NC * NS
CHUNK = 128
LANES = 16
NBUF = 8

B_TC = 8192
B_SC = B - B_TC
CHUNKS_PER_W = B_SC // (NW * CHUNK)
TC_BATCH = 128
TC_NBATCH = B_TC // TC_BATCH


def _tanh16(x):
    return 1.0 - 2.0 / (1.0 + jnp.exp(x * 2.0))


def _sc_body(idx_hbm, table_t_hbm, out_t_hbm, idx_v, bufs_v, out_v, sems):
    wid = lax.axis_index("s") * NC + lax.axis_index("c")
    chunk0 = wid * CHUNKS_PER_W
    iota = lax.iota(jnp.int32, LANES)

    pltpu.sync_copy(idx_hbm.at[pl.ds(chunk0, CHUNKS_PER_W)], idx_v)

    def issue(s, r):
        off = pl.multiple_of((r >> 7) * CHUNK, CHUNK)
        for a in range(0, D, 8):
            pltpu.make_async_copy(
                table_t_hbm.at[pl.ds(a, 8), pl.ds(off, CHUNK)],
                bufs_v.at[s, pl.ds(a, 8)],
                sems.at[s],
            ).start()

    def extract(s, r, col):
        m = jnp.full((LANES,), r & 127, jnp.int32)
        colv = jnp.full((LANES,), col, jnp.int32)
        for c0 in range(0, D, LANES):
            x = plsc.load_gather(bufs_v.at[s], [iota + c0, m])
            plsc.store_scatter(out_v, [iota + c0, colv], _tanh16(x))

    def drain(s):
        pltpu.make_async_copy(
            table_t_hbm.at[:, pl.ds(0, CHUNK)], bufs_v.at[s], sems.at[s]
        ).wait()

    for j in range(CHUNKS_PER_W):
        def group_body(h, idx_prev, j=j):
            idx16 = idx_v[j, pl.ds(h * LANES, LANES)]

            @pl.when(h > 0)
            def _():
                for s in range(NBUF):
                    drain(s)
                    extract(s, idx_prev[NBUF + s], (h - 1) * LANES + NBUF + s)

            for s in range(NBUF):
                issue(s, idx16[s])
            for s in range(NBUF):
                drain(s)
                extract(s, idx16[s], h * LANES + s)
            for s in range(NBUF):
                issue(s, idx16[NBUF + s])
            return idx16

        idx_last = lax.fori_loop(
            0, CHUNK // LANES, group_body, jnp.zeros((LANES,), jnp.int32)
        )
        for s in range(NBUF):
            drain(s)
            extract(s, idx_last[NBUF + s], CHUNK - NBUF + s)

        colbase = pl.multiple_of((chunk0 + j) * CHUNK, CHUNK)
        pltpu.sync_copy(out_v, out_t_hbm.at[:, pl.ds(colbase, CHUNK)])


def _run_sc(topics2d, table_t):
    mesh = plsc.VectorSubcoreMesh(core_axis_name="c", subcore_axis_name="s")
    f = functools.partial(
        pl.kernel,
        out_type=jax.ShapeDtypeStruct((D, B_SC), jnp.float32),
        mesh=mesh,
        scratch_types=[
            pltpu.VMEM((CHUNKS_PER_W, CHUNK), jnp.int32),
            pltpu.VMEM((NBUF, D, CHUNK), jnp.float32),
            pltpu.VMEM((D, CHUNK), jnp.float32),
            pltpu.SemaphoreType.DMA((NBUF,)),
        ],
        compiler_params=pltpu.CompilerParams(needs_layout_passes=False),
    )(_sc_body)
    return f(topics2d, table_t)


def _tc_body(idx_s, idx_v_ref, table_t_hbm, tail_hbm, out_ref, bufs_v, sems):
    lane_iota = lax.broadcasted_iota(jnp.int32, (TC_BATCH, CHUNK), 1)
    last_q = (V - 1) >> 7

    def issue_batch(k, kmod):
        def issue_one(i, _):
            r = idx_s[k * TC_BATCH + i]
            q = r >> 7
            off = pl.multiple_of(q * CHUNK, CHUNK)

            @pl.when(q < last_q)
            def _():
                pltpu.make_async_copy(
                    table_t_hbm.at[:, pl.ds(off, CHUNK)],
                    bufs_v.at[kmod, i],
                    sems.at[kmod],
                ).start()

            @pl.when(q >= last_q)
            def _():
                pltpu.make_async_copy(
                    tail_hbm, bufs_v.at[kmod, i], sems.at[kmod]
                ).start()

            return 0

        lax.fori_loop(0, TC_BATCH, issue_one, 0, unroll=8)

    def drain_batch(kmod):
        pltpu.make_async_copy(
            bufs_v.at[kmod], bufs_v.at[kmod], sems.at[kmod]
        ).wait()

    issue_batch(0, 0)

    def batch_body(k, _):
        kmod = lax.rem(k, 2)

        @pl.when(k < TC_NBATCH - 1)
        def _():
            issue_batch(k + 1, lax.rem(k + 1, 2))

        drain_batch(kmod)
        g = bufs_v[kmod]
        m = idx_v_ref[pl.ds(k * TC_BATCH, TC_BATCH)] & 127
        onehot = (lane_iota == m[:, None]).astype(jnp.float32)
        e = jnp.sum(g * onehot[:, None, :], axis=2)
        out_ref[:, pl.ds(k * TC_BATCH, TC_BATCH)] = jnp.tanh(e).T
        return 0

    lax.fori_loop(0, TC_NBATCH, batch_body, 0)


def _run_tc(topics_tc, table_t, tail):
    return pl.pallas_call(
        _tc_body,
        out_shape=jax.ShapeDtypeStruct((D, B_TC), jnp.float32),
        in_specs=[
            pl.BlockSpec(memory_space=pltpu.SMEM),
            pl.BlockSpec(memory_space=pltpu.VMEM),
            pl.BlockSpec(memory_space=pl.ANY),
            pl.BlockSpec(memory_space=pl.ANY),
        ],
        out_specs=pl.BlockSpec(memory_space=pltpu.VMEM),
        scratch_shapes=[
            pltpu.VMEM((2, TC_BATCH, D, CHUNK), jnp.float32),
            pltpu.SemaphoreType.DMA((2,)),
        ],
    )(topics_tc, topics_tc, table_t, tail)


@jax.jit
def _run(topics, table_t):
    topics_sc = topics[B_TC:].reshape(B_SC // CHUNK, CHUNK)
    last = (V >> 7) * CHUNK
    tail = jnp.pad(table_t[:, last:], ((0, 0), (0, CHUNK - (V - last))))
    out_sc = _run_sc(topics_sc, table_t)
    out_tc = _run_tc(topics[:B_TC], table_t, tail)
    return jnp.concatenate([out_tc, out_sc], axis=1)


def kernel(topics, table):
    out_t = _run(topics.astype(jnp.int32), table.T)
    return out_t.T.reshape(B, 1, D)
